# baseline (speedup 1.0000x reference)
import functools

import jax
import jax.numpy as jnp
from jax import lax
from jax.experimental import pallas as pl
from jax.experimental.pallas import tpu as pltpu
from jax.experimental.pallas import tpu_sc as plsc

N = 10000
E = 160000
B = 1024
EMBED = 128
GNN_IN = 256
HID = 64
PATH = 128
NCLS = 151

NC = 2
NS = 16
NW = NC * NS
CI = 8000
NCI = E // CI
GP = CI // 16


def _sc_scatter_add(qT3, src, dst):
    mesh = plsc.VectorSubcoreMesh(core_axis_name="c", subcore_axis_name="s")

    @functools.partial(
        pl.kernel,
        out_type=jax.ShapeDtypeStruct((NW, 2, N), jnp.float32),
        mesh=mesh,
        compiler_params=pltpu.CompilerParams(needs_layout_passes=False),
        scratch_types=[
            pltpu.VMEM((2, N), jnp.float32),
            pltpu.VMEM((2, N), jnp.float32),
            pltpu.VMEM((CI,), jnp.int32),
            pltpu.VMEM((CI,), jnp.int32),
            pltpu.VMEM((CI,), jnp.int32),
            pltpu.VMEM((CI,), jnp.int32),
            pltpu.SemaphoreType.DMA,
            pltpu.SemaphoreType.DMA,
        ],
    )
    def k(q_hbm, src_hbm, dst_hbm, out_hbm, qt_v, acc_v, sbuf0, sbuf1, dbuf0,
          dbuf1, semA, semB):
        sbufs = (sbuf0, sbuf1)
        dbufs = (dbuf0, dbuf1)
        cid = lax.axis_index("c")
        sid = lax.axis_index("s")
        wid = sid * NC + cid
        pltpu.sync_copy(q_hbm.at[wid], qt_v)

        zero16 = jnp.zeros((16,), jnp.float32)

        @plsc.parallel_loop(0, N // 16, unroll=8)
        def _(i):
            acc_v[0, pl.ds(i * 16, 16)] = zero16
            acc_v[1, pl.ds(i * 16, 16)] = zero16

        r0 = jnp.zeros((16,), jnp.int32)
        r1 = jnp.ones((16,), jnp.int32)

        pltpu.sync_copy(src_hbm.at[pl.ds(0, CI)], sbufs[0])
        pltpu.sync_copy(dst_hbm.at[pl.ds(0, CI)], dbufs[0])
        for c in range(NCI):
            b = c % 2
            if c + 1 < NCI:
                d1 = pltpu.async_copy(src_hbm.at[pl.ds((c + 1) * CI, CI)],
                                      sbufs[1 - b], semA)
                d2 = pltpu.async_copy(dst_hbm.at[pl.ds((c + 1) * CI, CI)],
                                      dbufs[1 - b], semB)

            @plsc.parallel_loop(0, GP, unroll=16)
            def _(g, sb=sbufs[b], db=dbufs[b]):
                s16 = sb[pl.ds(g * 16, 16)]
                d16 = db[pl.ds(g * 16, 16)]
                v0 = plsc.load_gather(qt_v, [r0, s16])
                plsc.addupdate_scatter(acc_v, [r0, d16], v0)
                v1 = plsc.load_gather(qt_v, [r1, s16])
                plsc.addupdate_scatter(acc_v, [r1, d16], v1)

            if c + 1 < NCI:
                d1.wait()
                d2.wait()
        pltpu.sync_copy(acc_v, out_hbm.at[wid])

    return k(qT3, src, dst)


def _scatter(q, src, dst):
    qT3 = q.T.reshape(NW, 2, N)
    agg3 = _sc_scatter_add(qT3, src, dst)
    return agg3.reshape(HID, N).T


def _tc_first_proj(x, W1):
    BN = 2000

    def body(x_ref, w_ref, o_ref):
        o_ref[...] = jnp.dot(x_ref[...], w_ref[...],
                             preferred_element_type=jnp.float32)

    return pl.pallas_call(
        body,
        grid=(N // BN,),
        in_specs=[
            pl.BlockSpec((BN, GNN_IN), lambda i: (i, 0)),
            pl.BlockSpec((GNN_IN, HID), lambda i: (0, 0)),
        ],
        out_specs=pl.BlockSpec((BN, HID), lambda i: (i, 0)),
        out_shape=jax.ShapeDtypeStruct((N, HID), jnp.float32),
    )(x, W1)


def _tc_mid_layer(q, agg, eps_i, b1, W2, b2, Wn):
    BN = 2000

    def body(q_ref, a_ref, e_ref, b1_ref, w2_ref, b2_ref, wn_ref, o_ref):
        u = (1.0 + e_ref[0, 0]) * q_ref[...] + a_ref[...] + b1_ref[...]
        t = jnp.maximum(u, 0.0)
        h = jnp.maximum(
            jnp.dot(t, w2_ref[...], preferred_element_type=jnp.float32)
            + b2_ref[...], 0.0)
        o_ref[...] = jnp.dot(h, wn_ref[...], preferred_element_type=jnp.float32)

    return pl.pallas_call(
        body,
        grid=(N // BN,),
        in_specs=[
            pl.BlockSpec((BN, HID), lambda i: (i, 0)),
            pl.BlockSpec((BN, HID), lambda i: (i, 0)),
            pl.BlockSpec((1, 1), lambda i: (0, 0)),
            pl.BlockSpec((1, HID), lambda i: (0, 0)),
            pl.BlockSpec((HID, HID), lambda i: (0, 0)),
            pl.BlockSpec((1, HID), lambda i: (0, 0)),
            pl.BlockSpec((HID, HID), lambda i: (0, 0)),
        ],
        out_specs=pl.BlockSpec((BN, HID), lambda i: (i, 0)),
        out_shape=jax.ShapeDtypeStruct((N, HID), jnp.float32),
    )(q, agg, eps_i, b1, W2, b2, Wn)


def _tc_last_layer_pool(q, agg, eps_i, b1, W2, b2, batch3):
    BN = 1000

    def body(q_ref, a_ref, e_ref, b1_ref, w2_ref, b2_ref, bt_ref, o_ref):
        i = pl.program_id(0)
        u = (1.0 + e_ref[0, 0]) * q_ref[...] + a_ref[...] + b1_ref[...]
        t = jnp.maximum(u, 0.0)
        h = jnp.maximum(
            jnp.dot(t, w2_ref[...], preferred_element_type=jnp.float32)
            + b2_ref[...], 0.0)
        seg = lax.broadcasted_iota(jnp.int32, (B, BN), 0)
        onehot = (seg == bt_ref[0]).astype(jnp.float32)
        contrib = jnp.dot(onehot, h, preferred_element_type=jnp.float32)

        @pl.when(i == 0)
        def _():
            o_ref[...] = contrib

        @pl.when(i > 0)
        def _():
            o_ref[...] += contrib

    return pl.pallas_call(
        body,
        grid=(N // BN,),
        in_specs=[
            pl.BlockSpec((BN, HID), lambda i: (i, 0)),
            pl.BlockSpec((BN, HID), lambda i: (i, 0)),
            pl.BlockSpec((1, 1), lambda i: (0, 0)),
            pl.BlockSpec((1, HID), lambda i: (0, 0)),
            pl.BlockSpec((HID, HID), lambda i: (0, 0)),
            pl.BlockSpec((1, HID), lambda i: (0, 0)),
            pl.BlockSpec((1, 1, BN), lambda i: (i, 0, 0)),
        ],
        out_specs=pl.BlockSpec((B, HID), lambda i: (0, 0)),
        out_shape=jax.ShapeDtypeStruct((B, HID), jnp.float32),
    )(q, agg, eps_i, b1, W2, b2, batch3)


def _tc_head(pooled, pred_pairs, entity_table, lin_W, lin_b, final_W, final_b):
    def body(p_ref, pp_ref, et_ref, lw_ref, lb_ref, fw_ref, fb_ref, o_ref):
        emb = jnp.dot(p_ref[...], lw_ref[...],
                      preferred_element_type=jnp.float32) + lb_ref[...]
        wf0 = fw_ref[0:EMBED, :]
        wf1 = fw_ref[EMBED:2 * EMBED, :]
        wf2 = fw_ref[2 * EMBED:2 * EMBED + PATH, :]
        e1 = jnp.dot(et_ref[...], wf1, preferred_element_type=jnp.float32)
        e2 = jnp.dot(et_ref[...], wf2, preferred_element_type=jnp.float32)
        cls_iota = lax.broadcasted_iota(jnp.int32, (B, NCLS), 1)
        oh0 = (cls_iota == pp_ref[:, 0:1]).astype(jnp.float32)
        oh1 = (cls_iota == pp_ref[:, 1:2]).astype(jnp.float32)
        o_ref[...] = (jnp.dot(emb, wf0, preferred_element_type=jnp.float32)
                      + jnp.dot(oh0, e1, preferred_element_type=jnp.float32)
                      + jnp.dot(oh1, e2, preferred_element_type=jnp.float32)
                      + fb_ref[...])

    return pl.pallas_call(
        body,
        out_shape=jax.ShapeDtypeStruct((B, PATH), jnp.float32),
    )(pooled, pred_pairs, entity_table, lin_W, lin_b.reshape(1, PATH),
      final_W, final_b.reshape(1, PATH))


def kernel(pred_pairs, union_features, x, edge_index, batch, word2neighbor,
           entity_table, eps, W1_0, b1_0, W2_0, b2_0, W1_1, b1_1, W2_1, b2_1,
           W1_2, b1_2, W2_2, b2_2, lin_W, lin_b, final_W, final_b):
    src = edge_index[0]
    dst = edge_index[1]
    batch3 = batch.reshape(N // 1000, 1, 1000)
    e0 = eps[0].reshape(1, 1)
    e1 = eps[1].reshape(1, 1)
    e2 = eps[2].reshape(1, 1)

    q0 = _tc_first_proj(x, W1_0)
    a0 = _scatter(q0, src, dst)
    q1 = _tc_mid_layer(q0, a0, e0, b1_0.reshape(1, HID), W2_0,
                       b2_0.reshape(1, HID), W1_1)
    a1 = _scatter(q1, src, dst)
    q2 = _tc_mid_layer(q1, a1, e1, b1_1.reshape(1, HID), W2_1,
                       b2_1.reshape(1, HID), W1_2)
    a2 = _scatter(q2, src, dst)
    pooled = _tc_last_layer_pool(q2, a2, e2, b1_2.reshape(1, HID),
                                 W2_2, b2_2.reshape(1, HID), batch3)
    return _tc_head(pooled, pred_pairs, entity_table, lin_W, lin_b, final_W,
                    final_b)


# speedup vs baseline: 1.0412x; 1.0412x over previous
import functools

import jax
import jax.numpy as jnp
from jax import lax
from jax.experimental import pallas as pl
from jax.experimental.pallas import tpu as pltpu
from jax.experimental.pallas import tpu_sc as plsc

N = 10000
E = 160000
B = 1024
EMBED = 128
GNN_IN = 256
HID = 64
PATH = 128
NCLS = 151

NC = 2
NS = 16
NW = NC * NS
CI = 8000
NCI = E // CI
GP = CI // 16


def _sc_scatter_add(qT3, src, dst):
    mesh = plsc.VectorSubcoreMesh(core_axis_name="c", subcore_axis_name="s")

    @functools.partial(
        pl.kernel,
        out_type=jax.ShapeDtypeStruct((NW, 2, N), jnp.float32),
        mesh=mesh,
        compiler_params=pltpu.CompilerParams(needs_layout_passes=False),
        scratch_types=[
            pltpu.VMEM((2, N), jnp.float32),
            pltpu.VMEM((2, N), jnp.float32),
            pltpu.VMEM((CI,), jnp.int32),
            pltpu.VMEM((CI,), jnp.int32),
            pltpu.VMEM((CI,), jnp.int32),
            pltpu.VMEM((CI,), jnp.int32),
            pltpu.SemaphoreType.DMA,
            pltpu.SemaphoreType.DMA,
        ],
    )
    def k(q_hbm, src_hbm, dst_hbm, out_hbm, qt_v, acc_v, sbuf0, sbuf1, dbuf0,
          dbuf1, semA, semB):
        sbufs = (sbuf0, sbuf1)
        dbufs = (dbuf0, dbuf1)
        cid = lax.axis_index("c")
        sid = lax.axis_index("s")
        wid = sid * NC + cid
        pltpu.sync_copy(q_hbm.at[wid], qt_v)

        zero16 = jnp.zeros((16,), jnp.float32)

        @plsc.parallel_loop(0, N // 16, unroll=8)
        def _(i):
            acc_v[0, pl.ds(i * 16, 16)] = zero16
            acc_v[1, pl.ds(i * 16, 16)] = zero16

        r0 = jnp.zeros((16,), jnp.int32)
        r1 = jnp.ones((16,), jnp.int32)

        pltpu.sync_copy(src_hbm.at[pl.ds(0, CI)], sbufs[0])
        pltpu.sync_copy(dst_hbm.at[pl.ds(0, CI)], dbufs[0])
        for c in range(NCI):
            b = c % 2
            if c + 1 < NCI:
                d1 = pltpu.async_copy(src_hbm.at[pl.ds((c + 1) * CI, CI)],
                                      sbufs[1 - b], semA)
                d2 = pltpu.async_copy(dst_hbm.at[pl.ds((c + 1) * CI, CI)],
                                      dbufs[1 - b], semB)

            @plsc.parallel_loop(0, GP, unroll=8)
            def _(g, sb=sbufs[b], db=dbufs[b]):
                s16 = sb[pl.ds(g * 16, 16)]
                d16 = db[pl.ds(g * 16, 16)]
                v0 = plsc.load_gather(qt_v, [r0, s16])
                plsc.addupdate_scatter(acc_v, [r0, d16], v0)
                v1 = plsc.load_gather(qt_v, [r1, s16])
                plsc.addupdate_scatter(acc_v, [r1, d16], v1)

            if c + 1 < NCI:
                d1.wait()
                d2.wait()
        pltpu.sync_copy(acc_v, out_hbm.at[wid])

    return k(qT3, src, dst)


def _scatter(q, src, dst):
    qT3 = q.T.reshape(NW, 2, N)
    agg3 = _sc_scatter_add(qT3, src, dst)
    return agg3.reshape(HID, N).T


def _tc_first_proj(x, W1):
    BN = 2000

    def body(x_ref, w_ref, o_ref):
        o_ref[...] = jnp.dot(x_ref[...], w_ref[...],
                             preferred_element_type=jnp.float32)

    return pl.pallas_call(
        body,
        grid=(N // BN,),
        in_specs=[
            pl.BlockSpec((BN, GNN_IN), lambda i: (i, 0)),
            pl.BlockSpec((GNN_IN, HID), lambda i: (0, 0)),
        ],
        out_specs=pl.BlockSpec((BN, HID), lambda i: (i, 0)),
        out_shape=jax.ShapeDtypeStruct((N, HID), jnp.float32),
    )(x, W1)


def _tc_mid_layer(q, agg, eps_i, b1, W2, b2, Wn):
    BN = 2000

    def body(q_ref, a_ref, e_ref, b1_ref, w2_ref, b2_ref, wn_ref, o_ref):
        u = (1.0 + e_ref[0, 0]) * q_ref[...] + a_ref[...] + b1_ref[...]
        t = jnp.maximum(u, 0.0)
        h = jnp.maximum(
            jnp.dot(t, w2_ref[...], preferred_element_type=jnp.float32)
            + b2_ref[...], 0.0)
        o_ref[...] = jnp.dot(h, wn_ref[...], preferred_element_type=jnp.float32)

    return pl.pallas_call(
        body,
        grid=(N // BN,),
        in_specs=[
            pl.BlockSpec((BN, HID), lambda i: (i, 0)),
            pl.BlockSpec((BN, HID), lambda i: (i, 0)),
            pl.BlockSpec((1, 1), lambda i: (0, 0)),
            pl.BlockSpec((1, HID), lambda i: (0, 0)),
            pl.BlockSpec((HID, HID), lambda i: (0, 0)),
            pl.BlockSpec((1, HID), lambda i: (0, 0)),
            pl.BlockSpec((HID, HID), lambda i: (0, 0)),
        ],
        out_specs=pl.BlockSpec((BN, HID), lambda i: (i, 0)),
        out_shape=jax.ShapeDtypeStruct((N, HID), jnp.float32),
    )(q, agg, eps_i, b1, W2, b2, Wn)


def _tc_last_layer_pool(q, agg, eps_i, b1, W2, b2, batch3):
    BN = 1000

    def body(q_ref, a_ref, e_ref, b1_ref, w2_ref, b2_ref, bt_ref, o_ref):
        i = pl.program_id(0)
        u = (1.0 + e_ref[0, 0]) * q_ref[...] + a_ref[...] + b1_ref[...]
        t = jnp.maximum(u, 0.0)
        h = jnp.maximum(
            jnp.dot(t, w2_ref[...], preferred_element_type=jnp.float32)
            + b2_ref[...], 0.0)
        seg = lax.broadcasted_iota(jnp.int32, (B, BN), 0)
        onehot = (seg == bt_ref[0]).astype(jnp.float32)
        contrib = jnp.dot(onehot, h, preferred_element_type=jnp.float32)

        @pl.when(i == 0)
        def _():
            o_ref[...] = contrib

        @pl.when(i > 0)
        def _():
            o_ref[...] += contrib

    return pl.pallas_call(
        body,
        grid=(N // BN,),
        in_specs=[
            pl.BlockSpec((BN, HID), lambda i: (i, 0)),
            pl.BlockSpec((BN, HID), lambda i: (i, 0)),
            pl.BlockSpec((1, 1), lambda i: (0, 0)),
            pl.BlockSpec((1, HID), lambda i: (0, 0)),
            pl.BlockSpec((HID, HID), lambda i: (0, 0)),
            pl.BlockSpec((1, HID), lambda i: (0, 0)),
            pl.BlockSpec((1, 1, BN), lambda i: (i, 0, 0)),
        ],
        out_specs=pl.BlockSpec((B, HID), lambda i: (0, 0)),
        out_shape=jax.ShapeDtypeStruct((B, HID), jnp.float32),
    )(q, agg, eps_i, b1, W2, b2, batch3)


def _tc_head(pooled, pred_pairs, entity_table, lin_W, lin_b, final_W, final_b):
    def body(p_ref, pp_ref, et_ref, lw_ref, lb_ref, fw_ref, fb_ref, o_ref):
        emb = jnp.dot(p_ref[...], lw_ref[...],
                      preferred_element_type=jnp.float32) + lb_ref[...]
        wf0 = fw_ref[0:EMBED, :]
        wf1 = fw_ref[EMBED:2 * EMBED, :]
        wf2 = fw_ref[2 * EMBED:2 * EMBED + PATH, :]
        e1 = jnp.dot(et_ref[...], wf1, preferred_element_type=jnp.float32)
        e2 = jnp.dot(et_ref[...], wf2, preferred_element_type=jnp.float32)
        cls_iota = lax.broadcasted_iota(jnp.int32, (B, NCLS), 1)
        oh0 = (cls_iota == pp_ref[:, 0:1]).astype(jnp.float32)
        oh1 = (cls_iota == pp_ref[:, 1:2]).astype(jnp.float32)
        o_ref[...] = (jnp.dot(emb, wf0, preferred_element_type=jnp.float32)
                      + jnp.dot(oh0, e1, preferred_element_type=jnp.float32)
                      + jnp.dot(oh1, e2, preferred_element_type=jnp.float32)
                      + fb_ref[...])

    return pl.pallas_call(
        body,
        out_shape=jax.ShapeDtypeStruct((B, PATH), jnp.float32),
    )(pooled, pred_pairs, entity_table, lin_W, lin_b.reshape(1, PATH),
      final_W, final_b.reshape(1, PATH))


def kernel(pred_pairs, union_features, x, edge_index, batch, word2neighbor,
           entity_table, eps, W1_0, b1_0, W2_0, b2_0, W1_1, b1_1, W2_1, b2_1,
           W1_2, b1_2, W2_2, b2_2, lin_W, lin_b, final_W, final_b):
    src = edge_index[0]
    dst = edge_index[1]
    batch3 = batch.reshape(N // 1000, 1, 1000)
    e0 = eps[0].reshape(1, 1)
    e1 = eps[1].reshape(1, 1)
    e2 = eps[2].reshape(1, 1)

    q0 = _tc_first_proj(x, W1_0)
    a0 = _scatter(q0, src, dst)
    q1 = _tc_mid_layer(q0, a0, e0, b1_0.reshape(1, HID), W2_0,
                       b2_0.reshape(1, HID), W1_1)
    a1 = _scatter(q1, src, dst)
    q2 = _tc_mid_layer(q1, a1, e1, b1_1.reshape(1, HID), W2_1,
                       b2_1.reshape(1, HID), W1_2)
    a2 = _scatter(q2, src, dst)
    pooled = _tc_last_layer_pool(q2, a2, e2, b1_2.reshape(1, HID),
                                 W2_2, b2_2.reshape(1, HID), batch3)
    return _tc_head(pooled, pred_pairs, entity_table, lin_W, lin_b, final_W,
                    final_b)
